# Initial kernel scaffold; baseline (speedup 1.0000x reference)
import jax
import jax.numpy as jnp
from jax.experimental import pallas as pl


def kernel(words, emit_W, emit_b, predcat_emb, root_W, root_b, rule_W, rule_b, op_W, op_b, s_in_W, s_in_b, r1_W1, r1_b1, r1_W2, r1_b2, r2_W1, r2_b1, r2_W2, r2_b2, s_out_W, s_out_b):
    raise NotImplementedError("write your pallas kernel here")


# speedup vs baseline: 2.3023x; 2.3023x over previous
import functools

import jax
import jax.numpy as jnp
from jax import lax
from jax.experimental import pallas as pl
from jax.experimental.pallas import tpu as pltpu
from jax.experimental.pallas import tpu_sc as plsc

STATE = 128
VOCAB = 100000
QALL = 300
B = 1024
L = 50
NWORDS = B * L

VC = 2048
NCHUNK = -(-VOCAB // VC)
VPAD = NCHUNK * VC

RBLK = 512
NRBLK = NWORDS // RBLK

_NEG = -1e30


def _lse_transpose_body(pred_ref, w_ref, b_ref, wt_ref, lse_ref, m_ref, s_ref):
    i = pl.program_id(0)
    w = w_ref[...]
    wt_ref[...] = w.T

    logits = jnp.dot(pred_ref[...], w, preferred_element_type=jnp.float32)
    logits = logits + b_ref[...]
    col = i * VC + lax.broadcasted_iota(jnp.int32, (1, VC), 1)
    lm = jnp.where(col < VOCAB, logits, _NEG)

    @pl.when(i == 0)
    def _():
        m_ref[...] = jnp.full((QALL, 1), _NEG, jnp.float32)
        s_ref[...] = jnp.zeros((QALL, 1), jnp.float32)

    m_old = m_ref[...]
    s_old = s_ref[...]
    m_new = jnp.maximum(m_old, jnp.max(lm, axis=1, keepdims=True))
    s_new = s_old * jnp.exp(m_old - m_new) + jnp.sum(
        jnp.exp(lm - m_new), axis=1, keepdims=True)
    m_ref[...] = m_new
    s_ref[...] = s_new

    @pl.when(i == NCHUNK - 1)
    def _():
        lse_ref[...] = m_new + jnp.log(s_new)


def _lse_transpose(predcat_emb, emit_W, emit_b2):
    return pl.pallas_call(
        _lse_transpose_body,
        grid=(NCHUNK,),
        in_specs=[
            pl.BlockSpec((QALL, STATE), lambda i: (0, 0)),
            pl.BlockSpec((STATE, VC), lambda i: (0, i)),
            pl.BlockSpec((1, VC), lambda i: (0, i)),
        ],
        out_specs=[
            pl.BlockSpec((VC, STATE), lambda i: (i, 0)),
            pl.BlockSpec((QALL, 1), lambda i: (0, 0)),
        ],
        out_shape=[
            jax.ShapeDtypeStruct((VPAD, STATE), jnp.float32),
            jax.ShapeDtypeStruct((QALL, 1), jnp.float32),
        ],
        scratch_shapes=[
            pltpu.VMEM((QALL, 1), jnp.float32),
            pltpu.VMEM((QALL, 1), jnp.float32),
        ],
    )(predcat_emb, emit_W, emit_b2)


def _sc_gather(wt, emit_b, words_flat):
    info = plsc.get_sparse_core_info()
    nc, ns = info.num_cores, info.num_subcores
    nw = nc * ns
    b_per_w = NWORDS // nw
    nchunk = 5
    ch = b_per_w // nchunk

    mesh = plsc.VectorSubcoreMesh(core_axis_name="c", subcore_axis_name="s")

    @functools.partial(
        pl.kernel, mesh=mesh,
        out_type=[
            jax.ShapeDtypeStruct((NWORDS, STATE), jnp.float32),
            jax.ShapeDtypeStruct((NWORDS,), jnp.float32),
        ],
        scratch_types=[
            pltpu.VMEM((b_per_w,), jnp.int32),
            pltpu.VMEM((ch, STATE), jnp.float32),
            pltpu.VMEM((ch,), jnp.float32),
            pltpu.SemaphoreType.DMA,
            pltpu.SemaphoreType.DMA,
        ],
    )
    def k(wt_hbm, b_hbm, words_hbm, g_hbm, bv_hbm, idx_v, rows_v, brow_v,
          sem_r, sem_b):
        wid = lax.axis_index("s") * nc + lax.axis_index("c")
        base = wid * b_per_w
        pltpu.sync_copy(words_hbm.at[pl.ds(base, b_per_w)], idx_v)
        for c in range(nchunk):
            idx_c = idx_v.at[pl.ds(c * ch, ch)]
            pltpu.async_copy(wt_hbm.at[idx_c], rows_v, sem_r).wait()
            pltpu.async_copy(b_hbm.at[idx_c], brow_v, sem_b).wait()
            pltpu.sync_copy(rows_v, g_hbm.at[pl.ds(base + c * ch, ch)])
            pltpu.sync_copy(brow_v, bv_hbm.at[pl.ds(base + c * ch, ch)])

    return k(wt, emit_b, words_flat)


def _log_softmax_rows(x):
    m = jnp.max(x, axis=1, keepdims=True)
    return x - m - jnp.log(jnp.sum(jnp.exp(x - m), axis=1, keepdims=True))


def _emit_body(g_ref, bv_ref, pred_ref, lse_ref,
               root_W_ref, root_b_ref, rule_W_ref, rule_b_ref,
               op_W_ref, op_b_ref, s_in_W_ref, s_in_b_ref,
               r1_W1_ref, r1_b1_ref, r1_W2_ref, r1_b2_ref,
               r2_W1_ref, r2_b1_ref, r2_W2_ref, r2_b2_ref,
               s_out_W_ref, s_out_b_ref,
               x_ref, root_ref, rule_ref, op_ref, split_ref):
    x = lax.dot_general(g_ref[...], pred_ref[...],
                        (((1,), (1,)), ((), ())),
                        preferred_element_type=jnp.float32)
    x_ref[...] = x + bv_ref[...] - lse_ref[...]

    @pl.when(pl.program_id(0) == 0)
    def _():
        root_ref[...] = _log_softmax_rows(root_W_ref[...] + root_b_ref[...])
        rule_ref[...] = _log_softmax_rows(rule_W_ref[...] + rule_b_ref[...])
        op_ref[...] = _log_softmax_rows(op_W_ref[...] + op_b_ref[...])
        pred = pred_ref[...]
        h = jnp.dot(pred, s_in_W_ref[...],
                    preferred_element_type=jnp.float32) + s_in_b_ref[...]
        t = jax.nn.relu(jnp.dot(h, r1_W1_ref[...],
                                preferred_element_type=jnp.float32)
                        + r1_b1_ref[...])
        h = h + jax.nn.relu(jnp.dot(t, r1_W2_ref[...],
                                    preferred_element_type=jnp.float32)
                            + r1_b2_ref[...])
        t = jax.nn.relu(jnp.dot(h, r2_W1_ref[...],
                                preferred_element_type=jnp.float32)
                        + r2_b1_ref[...])
        h = h + jax.nn.relu(jnp.dot(t, r2_W2_ref[...],
                                    preferred_element_type=jnp.float32)
                            + r2_b2_ref[...])
        sp = jnp.dot(h, s_out_W_ref[...],
                     preferred_element_type=jnp.float32) + s_out_b_ref[...]
        split_ref[...] = _log_softmax_rows(sp)


def _emit(g, bv2, predcat_emb, lse2, root_W, root_b2, rule_W, rule_b2,
          op_W, op_b2, s_in_W, s_in_b2, r1_W1, r1_b12, r1_W2, r1_b22,
          r2_W1, r2_b12, r2_W2, r2_b22, s_out_W, s_out_b2):
    full = lambda shape: pl.BlockSpec(shape, lambda i: (0,) * len(shape))
    return pl.pallas_call(
        _emit_body,
        grid=(NRBLK,),
        in_specs=[
            pl.BlockSpec((RBLK, STATE), lambda i: (i, 0)),
            pl.BlockSpec((RBLK, 1), lambda i: (i, 0)),
            full((QALL, STATE)),
            full((1, QALL)),
            full((1, QALL)), full((1, QALL)),
            full(rule_W.shape), full((1, rule_W.shape[1])),
            full(op_W.shape), full((1, op_W.shape[1])),
            full(s_in_W.shape), full((1, STATE)),
            full(r1_W1.shape), full((1, STATE)),
            full(r1_W2.shape), full((1, STATE)),
            full(r2_W1.shape), full((1, STATE)),
            full(r2_W2.shape), full((1, STATE)),
            full(s_out_W.shape), full((1, s_out_W.shape[1])),
        ],
        out_specs=[
            pl.BlockSpec((RBLK, QALL), lambda i: (i, 0)),
            full((1, QALL)),
            full(rule_W.shape),
            full(op_W.shape),
            full((QALL, s_out_W.shape[1])),
        ],
        out_shape=[
            jax.ShapeDtypeStruct((NWORDS, QALL), jnp.float32),
            jax.ShapeDtypeStruct((1, QALL), jnp.float32),
            jax.ShapeDtypeStruct(rule_W.shape, jnp.float32),
            jax.ShapeDtypeStruct(op_W.shape, jnp.float32),
            jax.ShapeDtypeStruct((QALL, s_out_W.shape[1]), jnp.float32),
        ],
    )(g, bv2, predcat_emb, lse2, root_W, root_b2, rule_W, rule_b2,
      op_W, op_b2, s_in_W, s_in_b2, r1_W1, r1_b12, r1_W2, r1_b22,
      r2_W1, r2_b12, r2_W2, r2_b22, s_out_W, s_out_b2)


def kernel(words, emit_W, emit_b, predcat_emb, root_W, root_b, rule_W, rule_b,
           op_W, op_b, s_in_W, s_in_b, r1_W1, r1_b1, r1_W2, r1_b2,
           r2_W1, r2_b1, r2_W2, r2_b2, s_out_W, s_out_b):
    wt, lse = _lse_transpose(predcat_emb, emit_W, emit_b.reshape(1, VOCAB))
    words_flat = words.reshape(NWORDS).astype(jnp.int32)
    g, bv = _sc_gather(wt, emit_b, words_flat)
    x, root, rule, op, split = _emit(
        g, bv.reshape(NWORDS, 1), predcat_emb, lse.reshape(1, QALL),
        root_W, root_b.reshape(1, QALL), rule_W, rule_b.reshape(1, -1),
        op_W, op_b.reshape(1, -1), s_in_W, s_in_b.reshape(1, -1),
        r1_W1, r1_b1.reshape(1, -1), r1_W2, r1_b2.reshape(1, -1),
        r2_W1, r2_b1.reshape(1, -1), r2_W2, r2_b2.reshape(1, -1),
        s_out_W, s_out_b.reshape(1, -1))
    return (x.reshape(B, L, QALL), root.reshape(QALL), rule, op, split)


# speedup vs baseline: 2.7595x; 1.1986x over previous
import functools

import jax
import jax.numpy as jnp
from jax import lax
from jax.experimental import pallas as pl
from jax.experimental.pallas import tpu as pltpu
from jax.experimental.pallas import tpu_sc as plsc

STATE = 128
VOCAB = 100000
QALL = 300
B = 1024
L = 50
NWORDS = B * L

VC = 2048
NCHUNK = -(-VOCAB // VC)
VPAD = NCHUNK * VC

SENT_BLK = 16
RBLK = SENT_BLK * L
NRBLK = NWORDS // RBLK

_NEG = -1e30


def _lse_transpose_body(pred_ref, w_ref, b_ref, wt_ref, lse_ref, m_ref, s_ref):
    i = pl.program_id(0)
    w = w_ref[...]
    wt_ref[...] = w.T

    logits = jnp.dot(pred_ref[...], w, preferred_element_type=jnp.float32)
    logits = logits + b_ref[...]
    col = i * VC + lax.broadcasted_iota(jnp.int32, (1, VC), 1)
    lm = jnp.where(col < VOCAB, logits, _NEG)

    @pl.when(i == 0)
    def _():
        m_ref[...] = jnp.full((QALL, 1), _NEG, jnp.float32)
        s_ref[...] = jnp.zeros((QALL, 1), jnp.float32)

    m_old = m_ref[...]
    s_old = s_ref[...]
    m_new = jnp.maximum(m_old, jnp.max(lm, axis=1, keepdims=True))
    s_new = s_old * jnp.exp(m_old - m_new) + jnp.sum(
        jnp.exp(lm - m_new), axis=1, keepdims=True)
    m_ref[...] = m_new
    s_ref[...] = s_new

    @pl.when(i == NCHUNK - 1)
    def _():
        lse_ref[...] = m_new + jnp.log(s_new)


def _lse_transpose(predcat_emb, emit_W, emit_b2):
    return pl.pallas_call(
        _lse_transpose_body,
        grid=(NCHUNK,),
        in_specs=[
            pl.BlockSpec((QALL, STATE), lambda i: (0, 0)),
            pl.BlockSpec((STATE, VC), lambda i: (0, i)),
            pl.BlockSpec((1, VC), lambda i: (0, i)),
        ],
        out_specs=[
            pl.BlockSpec((VC, STATE), lambda i: (i, 0)),
            pl.BlockSpec((QALL, 1), lambda i: (0, 0)),
        ],
        out_shape=[
            jax.ShapeDtypeStruct((VPAD, STATE), jnp.float32),
            jax.ShapeDtypeStruct((QALL, 1), jnp.float32),
        ],
        scratch_shapes=[
            pltpu.VMEM((QALL, 1), jnp.float32),
            pltpu.VMEM((QALL, 1), jnp.float32),
        ],
    )(predcat_emb, emit_W, emit_b2)


def _sc_gather(wt, emit_b, words_flat):
    info = plsc.get_sparse_core_info()
    nc, ns = info.num_cores, info.num_subcores
    nw = nc * ns
    b_per_w = NWORDS // nw
    nchunk = 5
    ch = b_per_w // nchunk

    mesh = plsc.VectorSubcoreMesh(core_axis_name="c", subcore_axis_name="s")

    @functools.partial(
        pl.kernel, mesh=mesh,
        out_type=[
            jax.ShapeDtypeStruct((NWORDS, STATE), jnp.float32),
            jax.ShapeDtypeStruct((NWORDS,), jnp.float32),
        ],
        scratch_types=[
            pltpu.VMEM((b_per_w,), jnp.int32),
            pltpu.VMEM((ch, STATE), jnp.float32),
            pltpu.VMEM((ch,), jnp.float32),
            pltpu.SemaphoreType.DMA,
            pltpu.SemaphoreType.DMA,
        ],
    )
    def k(wt_hbm, b_hbm, words_hbm, g_hbm, bv_hbm, idx_v, rows_v, brow_v,
          sem_r, sem_b):
        wid = lax.axis_index("s") * nc + lax.axis_index("c")
        base = wid * b_per_w
        pltpu.sync_copy(words_hbm.at[pl.ds(base, b_per_w)], idx_v)
        for c in range(nchunk):
            idx_c = idx_v.at[pl.ds(c * ch, ch)]
            pltpu.async_copy(wt_hbm.at[idx_c], rows_v, sem_r).wait()
            pltpu.async_copy(b_hbm.at[idx_c], brow_v, sem_b).wait()
            pltpu.sync_copy(rows_v, g_hbm.at[pl.ds(base + c * ch, ch)])
            pltpu.sync_copy(brow_v, bv_hbm.at[pl.ds(base + c * ch, ch)])

    return k(wt, emit_b, words_flat)


def _log_softmax_rows(x):
    m = jnp.max(x, axis=1, keepdims=True)
    return x - m - jnp.log(jnp.sum(jnp.exp(x - m), axis=1, keepdims=True))


def _emit_body(g_ref, bv_ref, pred_ref, lse_ref,
               root_W_ref, root_b_ref, rule_W_ref, rule_b_ref,
               op_W_ref, op_b_ref, s_in_W_ref, s_in_b_ref,
               r1_W1_ref, r1_b1_ref, r1_W2_ref, r1_b2_ref,
               r2_W1_ref, r2_b1_ref, r2_W2_ref, r2_b2_ref,
               s_out_W_ref, s_out_b_ref,
               x_ref, root_ref, rule_ref, op_ref, split_ref):
    x = lax.dot_general(g_ref[...], pred_ref[...],
                        (((1,), (1,)), ((), ())),
                        preferred_element_type=jnp.float32)
    x = x + bv_ref[...] - lse_ref[...]
    x_ref[...] = x.reshape(SENT_BLK, L, QALL)

    @pl.when(pl.program_id(0) == 0)
    def _():
        root_ref[...] = _log_softmax_rows(root_W_ref[...] + root_b_ref[...])
        rule_ref[...] = _log_softmax_rows(rule_W_ref[...] + rule_b_ref[...])
        op_ref[...] = _log_softmax_rows(op_W_ref[...] + op_b_ref[...])
        pred = pred_ref[...]
        h = jnp.dot(pred, s_in_W_ref[...],
                    preferred_element_type=jnp.float32) + s_in_b_ref[...]
        t = jax.nn.relu(jnp.dot(h, r1_W1_ref[...],
                                preferred_element_type=jnp.float32)
                        + r1_b1_ref[...])
        h = h + jax.nn.relu(jnp.dot(t, r1_W2_ref[...],
                                    preferred_element_type=jnp.float32)
                            + r1_b2_ref[...])
        t = jax.nn.relu(jnp.dot(h, r2_W1_ref[...],
                                preferred_element_type=jnp.float32)
                        + r2_b1_ref[...])
        h = h + jax.nn.relu(jnp.dot(t, r2_W2_ref[...],
                                    preferred_element_type=jnp.float32)
                            + r2_b2_ref[...])
        sp = jnp.dot(h, s_out_W_ref[...],
                     preferred_element_type=jnp.float32) + s_out_b_ref[...]
        split_ref[...] = _log_softmax_rows(sp)


def _emit(g, bv2, predcat_emb, lse2, root_W, root_b2, rule_W, rule_b2,
          op_W, op_b2, s_in_W, s_in_b2, r1_W1, r1_b12, r1_W2, r1_b22,
          r2_W1, r2_b12, r2_W2, r2_b22, s_out_W, s_out_b2):
    full = lambda shape: pl.BlockSpec(shape, lambda i: (0,) * len(shape))
    return pl.pallas_call(
        _emit_body,
        grid=(NRBLK,),
        in_specs=[
            pl.BlockSpec((RBLK, STATE), lambda i: (i, 0)),
            pl.BlockSpec((RBLK, 1), lambda i: (i, 0)),
            full((QALL, STATE)),
            full((1, QALL)),
            full((1, QALL)), full((1, QALL)),
            full(rule_W.shape), full((1, rule_W.shape[1])),
            full(op_W.shape), full((1, op_W.shape[1])),
            full(s_in_W.shape), full((1, STATE)),
            full(r1_W1.shape), full((1, STATE)),
            full(r1_W2.shape), full((1, STATE)),
            full(r2_W1.shape), full((1, STATE)),
            full(r2_W2.shape), full((1, STATE)),
            full(s_out_W.shape), full((1, s_out_W.shape[1])),
        ],
        out_specs=[
            pl.BlockSpec((SENT_BLK, L, QALL), lambda i: (i, 0, 0)),
            full((1, QALL)),
            full(rule_W.shape),
            full(op_W.shape),
            full((QALL, s_out_W.shape[1])),
        ],
        out_shape=[
            jax.ShapeDtypeStruct((B, L, QALL), jnp.float32),
            jax.ShapeDtypeStruct((1, QALL), jnp.float32),
            jax.ShapeDtypeStruct(rule_W.shape, jnp.float32),
            jax.ShapeDtypeStruct(op_W.shape, jnp.float32),
            jax.ShapeDtypeStruct((QALL, s_out_W.shape[1]), jnp.float32),
        ],
    )(g, bv2, predcat_emb, lse2, root_W, root_b2, rule_W, rule_b2,
      op_W, op_b2, s_in_W, s_in_b2, r1_W1, r1_b12, r1_W2, r1_b22,
      r2_W1, r2_b12, r2_W2, r2_b22, s_out_W, s_out_b2)


def kernel(words, emit_W, emit_b, predcat_emb, root_W, root_b, rule_W, rule_b,
           op_W, op_b, s_in_W, s_in_b, r1_W1, r1_b1, r1_W2, r1_b2,
           r2_W1, r2_b1, r2_W2, r2_b2, s_out_W, s_out_b):
    wt, lse = _lse_transpose(predcat_emb, emit_W, emit_b.reshape(1, VOCAB))
    words_flat = words.reshape(NWORDS).astype(jnp.int32)
    g, bv = _sc_gather(wt, emit_b, words_flat)
    x, root, rule, op, split = _emit(
        g, bv.reshape(NWORDS, 1), predcat_emb, lse.reshape(1, QALL),
        root_W, root_b.reshape(1, QALL), rule_W, rule_b.reshape(1, -1),
        op_W, op_b.reshape(1, -1), s_in_W, s_in_b.reshape(1, -1),
        r1_W1, r1_b1.reshape(1, -1), r1_W2, r1_b2.reshape(1, -1),
        r2_W1, r2_b1.reshape(1, -1), r2_W2, r2_b2.reshape(1, -1),
        s_out_W, s_out_b.reshape(1, -1))
    return (x, root.reshape(QALL), rule, op, split)


# speedup vs baseline: 2.7608x; 1.0005x over previous
import functools

import jax
import jax.numpy as jnp
from jax import lax
from jax.experimental import pallas as pl
from jax.experimental.pallas import tpu as pltpu
from jax.experimental.pallas import tpu_sc as plsc

STATE = 128
VOCAB = 100000
QALL = 300
B = 1024
L = 50
NWORDS = B * L

VC = 2048
NCHUNK = -(-VOCAB // VC)
VPAD = NCHUNK * VC

SENT_BLK = 16
RBLK = SENT_BLK * L
NRBLK = NWORDS // RBLK

_NEG = -1e30


def _lse_transpose_body(pred_ref, w_ref, b_ref, wt_ref, lse_ref, m_ref, s_ref):
    i = pl.program_id(0)
    w = w_ref[...]
    wt_ref[...] = w.T

    logits = jnp.dot(pred_ref[...], w, preferred_element_type=jnp.float32)
    logits = logits + b_ref[...]
    col = i * VC + lax.broadcasted_iota(jnp.int32, (1, VC), 1)
    lm = jnp.where(col < VOCAB, logits, _NEG)

    @pl.when(i == 0)
    def _():
        m_ref[...] = jnp.full((QALL, 1), _NEG, jnp.float32)
        s_ref[...] = jnp.zeros((QALL, 1), jnp.float32)

    m_old = m_ref[...]
    s_old = s_ref[...]
    m_new = jnp.maximum(m_old, jnp.max(lm, axis=1, keepdims=True))
    s_new = s_old * jnp.exp(m_old - m_new) + jnp.sum(
        jnp.exp(lm - m_new), axis=1, keepdims=True)
    m_ref[...] = m_new
    s_ref[...] = s_new

    @pl.when(i == NCHUNK - 1)
    def _():
        lse_ref[...] = m_new + jnp.log(s_new)


def _lse_transpose(predcat_emb, emit_W, emit_b2):
    return pl.pallas_call(
        _lse_transpose_body,
        grid=(NCHUNK,),
        in_specs=[
            pl.BlockSpec((QALL, STATE), lambda i: (0, 0)),
            pl.BlockSpec((STATE, VC), lambda i: (0, i)),
            pl.BlockSpec((1, VC), lambda i: (0, i)),
        ],
        out_specs=[
            pl.BlockSpec((VC, STATE), lambda i: (i, 0)),
            pl.BlockSpec((QALL, 1), lambda i: (0, 0)),
        ],
        out_shape=[
            jax.ShapeDtypeStruct((VPAD, STATE), jnp.float32),
            jax.ShapeDtypeStruct((QALL, 1), jnp.float32),
        ],
        scratch_shapes=[
            pltpu.VMEM((QALL, 1), jnp.float32),
            pltpu.VMEM((QALL, 1), jnp.float32),
        ],
    )(predcat_emb, emit_W, emit_b2)


def _sc_gather(wt, emit_b, words_flat):
    info = plsc.get_sparse_core_info()
    nc, ns = info.num_cores, info.num_subcores
    nw = nc * ns
    b_per_w = NWORDS // nw
    nchunk = 5
    ch = b_per_w // nchunk

    mesh = plsc.VectorSubcoreMesh(core_axis_name="c", subcore_axis_name="s")

    @functools.partial(
        pl.kernel, mesh=mesh,
        compiler_params=pltpu.CompilerParams(use_tc_tiling_on_sc=True),
        out_type=[
            jax.ShapeDtypeStruct((NWORDS, STATE), jnp.float32),
            jax.ShapeDtypeStruct((NWORDS,), jnp.float32),
        ],
        scratch_types=[
            pltpu.VMEM((b_per_w,), jnp.int32),
            pltpu.VMEM((ch, STATE), jnp.float32),
            pltpu.VMEM((ch,), jnp.float32),
            pltpu.SemaphoreType.DMA,
            pltpu.SemaphoreType.DMA,
        ],
    )
    def k(wt_hbm, b_hbm, words_hbm, g_hbm, bv_hbm, idx_v, rows_v, brow_v,
          sem_r, sem_b):
        wid = lax.axis_index("s") * nc + lax.axis_index("c")
        base = wid * b_per_w
        pltpu.sync_copy(words_hbm.at[pl.ds(base, b_per_w)], idx_v)
        for c in range(nchunk):
            idx_c = idx_v.at[pl.ds(c * ch, ch)]
            pltpu.async_copy(wt_hbm.at[idx_c], rows_v, sem_r).wait()
            pltpu.async_copy(b_hbm.at[idx_c], brow_v, sem_b).wait()
            pltpu.sync_copy(rows_v, g_hbm.at[pl.ds(base + c * ch, ch)])
            pltpu.sync_copy(brow_v, bv_hbm.at[pl.ds(base + c * ch, ch)])

    return k(wt, emit_b, words_flat)


def _log_softmax_rows(x):
    m = jnp.max(x, axis=1, keepdims=True)
    return x - m - jnp.log(jnp.sum(jnp.exp(x - m), axis=1, keepdims=True))


def _emit_body(g_ref, bv_ref, pred_ref, lse_ref,
               root_W_ref, root_b_ref, rule_W_ref, rule_b_ref,
               op_W_ref, op_b_ref, s_in_W_ref, s_in_b_ref,
               r1_W1_ref, r1_b1_ref, r1_W2_ref, r1_b2_ref,
               r2_W1_ref, r2_b1_ref, r2_W2_ref, r2_b2_ref,
               s_out_W_ref, s_out_b_ref,
               x_ref, root_ref, rule_ref, op_ref, split_ref):
    x = lax.dot_general(g_ref[...], pred_ref[...],
                        (((1,), (1,)), ((), ())),
                        preferred_element_type=jnp.float32)
    x = x + bv_ref[...] - lse_ref[...]
    x_ref[...] = x.reshape(SENT_BLK, L, QALL)

    @pl.when(pl.program_id(0) == 0)
    def _():
        root_ref[...] = _log_softmax_rows(root_W_ref[...] + root_b_ref[...])
        rule_ref[...] = _log_softmax_rows(rule_W_ref[...] + rule_b_ref[...])
        op_ref[...] = _log_softmax_rows(op_W_ref[...] + op_b_ref[...])
        pred = pred_ref[...]
        h = jnp.dot(pred, s_in_W_ref[...],
                    preferred_element_type=jnp.float32) + s_in_b_ref[...]
        t = jax.nn.relu(jnp.dot(h, r1_W1_ref[...],
                                preferred_element_type=jnp.float32)
                        + r1_b1_ref[...])
        h = h + jax.nn.relu(jnp.dot(t, r1_W2_ref[...],
                                    preferred_element_type=jnp.float32)
                            + r1_b2_ref[...])
        t = jax.nn.relu(jnp.dot(h, r2_W1_ref[...],
                                preferred_element_type=jnp.float32)
                        + r2_b1_ref[...])
        h = h + jax.nn.relu(jnp.dot(t, r2_W2_ref[...],
                                    preferred_element_type=jnp.float32)
                            + r2_b2_ref[...])
        sp = jnp.dot(h, s_out_W_ref[...],
                     preferred_element_type=jnp.float32) + s_out_b_ref[...]
        split_ref[...] = _log_softmax_rows(sp)


def _emit(g, bv2, predcat_emb, lse2, root_W, root_b2, rule_W, rule_b2,
          op_W, op_b2, s_in_W, s_in_b2, r1_W1, r1_b12, r1_W2, r1_b22,
          r2_W1, r2_b12, r2_W2, r2_b22, s_out_W, s_out_b2):
    full = lambda shape: pl.BlockSpec(shape, lambda i: (0,) * len(shape))
    return pl.pallas_call(
        _emit_body,
        grid=(NRBLK,),
        in_specs=[
            pl.BlockSpec((RBLK, STATE), lambda i: (i, 0)),
            pl.BlockSpec((RBLK, 1), lambda i: (i, 0)),
            full((QALL, STATE)),
            full((1, QALL)),
            full((1, QALL)), full((1, QALL)),
            full(rule_W.shape), full((1, rule_W.shape[1])),
            full(op_W.shape), full((1, op_W.shape[1])),
            full(s_in_W.shape), full((1, STATE)),
            full(r1_W1.shape), full((1, STATE)),
            full(r1_W2.shape), full((1, STATE)),
            full(r2_W1.shape), full((1, STATE)),
            full(r2_W2.shape), full((1, STATE)),
            full(s_out_W.shape), full((1, s_out_W.shape[1])),
        ],
        out_specs=[
            pl.BlockSpec((SENT_BLK, L, QALL), lambda i: (i, 0, 0)),
            full((1, QALL)),
            full(rule_W.shape),
            full(op_W.shape),
            full((QALL, s_out_W.shape[1])),
        ],
        out_shape=[
            jax.ShapeDtypeStruct((B, L, QALL), jnp.float32),
            jax.ShapeDtypeStruct((1, QALL), jnp.float32),
            jax.ShapeDtypeStruct(rule_W.shape, jnp.float32),
            jax.ShapeDtypeStruct(op_W.shape, jnp.float32),
            jax.ShapeDtypeStruct((QALL, s_out_W.shape[1]), jnp.float32),
        ],
    )(g, bv2, predcat_emb, lse2, root_W, root_b2, rule_W, rule_b2,
      op_W, op_b2, s_in_W, s_in_b2, r1_W1, r1_b12, r1_W2, r1_b22,
      r2_W1, r2_b12, r2_W2, r2_b22, s_out_W, s_out_b2)


def kernel(words, emit_W, emit_b, predcat_emb, root_W, root_b, rule_W, rule_b,
           op_W, op_b, s_in_W, s_in_b, r1_W1, r1_b1, r1_W2, r1_b2,
           r2_W1, r2_b1, r2_W2, r2_b2, s_out_W, s_out_b):
    wt, lse = _lse_transpose(predcat_emb, emit_W, emit_b.reshape(1, VOCAB))
    words_flat = words.reshape(NWORDS).astype(jnp.int32)
    g, bv = _sc_gather(wt, emit_b, words_flat)
    x, root, rule, op, split = _emit(
        g, bv.reshape(NWORDS, 1), predcat_emb, lse.reshape(1, QALL),
        root_W, root_b.reshape(1, QALL), rule_W, rule_b.reshape(1, -1),
        op_W, op_b.reshape(1, -1), s_in_W, s_in_b.reshape(1, -1),
        r1_W1, r1_b1.reshape(1, -1), r1_W2, r1_b2.reshape(1, -1),
        r2_W1, r2_b1.reshape(1, -1), r2_W2, r2_b2.reshape(1, -1),
        s_out_W, s_out_b.reshape(1, -1))
    return (x, root.reshape(QALL), rule, op, split)


# speedup vs baseline: 6.2346x; 2.2582x over previous
import functools

import jax
import jax.numpy as jnp
from jax import lax
from jax.experimental import pallas as pl
from jax.experimental.pallas import tpu as pltpu
from jax.experimental.pallas import tpu_sc as plsc

STATE = 128
VOCAB = 100000
QALL = 300
B = 1024
L = 50
NWORDS = B * L

VC = 2048
NCHUNK = -(-VOCAB // VC)

_NEG = -1e30


def _lse_body(pred_ref, wt_ref, b_ref, lse_ref, m_ref, s_ref):
    i = pl.program_id(0)
    logits = lax.dot_general(pred_ref[...], wt_ref[...],
                             (((1,), (1,)), ((), ())),
                             preferred_element_type=jnp.float32)
    logits = logits + b_ref[...]
    col = i * VC + lax.broadcasted_iota(jnp.int32, (1, VC), 1)
    lm = jnp.where(col < VOCAB, logits, _NEG)

    @pl.when(i == 0)
    def _():
        m_ref[...] = jnp.full((QALL, 1), _NEG, jnp.float32)
        s_ref[...] = jnp.zeros((QALL, 1), jnp.float32)

    m_old = m_ref[...]
    s_old = s_ref[...]
    m_new = jnp.maximum(m_old, jnp.max(lm, axis=1, keepdims=True))
    s_new = s_old * jnp.exp(m_old - m_new) + jnp.sum(
        jnp.exp(lm - m_new), axis=1, keepdims=True)
    m_ref[...] = m_new
    s_ref[...] = s_new

    @pl.when(i == NCHUNK - 1)
    def _():
        lse_ref[...] = m_new + jnp.log(s_new)


def _lse(predcat_emb, emit_wt, emit_b2):
    return pl.pallas_call(
        _lse_body,
        grid=(NCHUNK,),
        in_specs=[
            pl.BlockSpec((QALL, STATE), lambda i: (0, 0)),
            pl.BlockSpec((VC, STATE), lambda i: (i, 0)),
            pl.BlockSpec((1, VC), lambda i: (0, i)),
        ],
        out_specs=pl.BlockSpec((QALL, 1), lambda i: (0, 0)),
        out_shape=jax.ShapeDtypeStruct((QALL, 1), jnp.float32),
        scratch_shapes=[
            pltpu.VMEM((QALL, 1), jnp.float32),
            pltpu.VMEM((QALL, 1), jnp.float32),
        ],
    )(predcat_emb, emit_wt, emit_b2)


def _sc_gather(wt, emit_b, words_flat):
    info = plsc.get_sparse_core_info()
    nc, ns = info.num_cores, info.num_subcores
    nw = nc * ns
    b_per_w = NWORDS // nw
    nchunk = 5
    ch = b_per_w // nchunk

    mesh = plsc.VectorSubcoreMesh(core_axis_name="c", subcore_axis_name="s")

    @functools.partial(
        pl.kernel, mesh=mesh,
        out_type=[
            jax.ShapeDtypeStruct((NWORDS, STATE), jnp.float32),
            jax.ShapeDtypeStruct((NWORDS,), jnp.float32),
        ],
        scratch_types=[
            pltpu.VMEM((b_per_w,), jnp.int32),
            pltpu.VMEM((ch, STATE), jnp.float32),
            pltpu.VMEM((ch,), jnp.float32),
            pltpu.SemaphoreType.DMA,
            pltpu.SemaphoreType.DMA,
        ],
    )
    def k(wt_hbm, b_hbm, words_hbm, g_hbm, bv_hbm, idx_v, rows_v, brow_v,
          sem_r, sem_b):
        wid = lax.axis_index("s") * nc + lax.axis_index("c")
        base = wid * b_per_w
        pltpu.sync_copy(words_hbm.at[pl.ds(base, b_per_w)], idx_v)
        for c in range(nchunk):
            idx_c = idx_v.at[pl.ds(c * ch, ch)]
            pltpu.async_copy(wt_hbm.at[idx_c], rows_v, sem_r).wait()
            pltpu.async_copy(b_hbm.at[idx_c], brow_v, sem_b).wait()
            pltpu.sync_copy(rows_v, g_hbm.at[pl.ds(base + c * ch, ch)])
            pltpu.sync_copy(brow_v, bv_hbm.at[pl.ds(base + c * ch, ch)])

    return k(wt, emit_b, words_flat)


def _log_softmax_rows(x):
    m = jnp.max(x, axis=1, keepdims=True)
    return x - m - jnp.log(jnp.sum(jnp.exp(x - m), axis=1, keepdims=True))


def _emit_body(g_ref, bv_ref, pred_ref, lse_ref,
               root_W_ref, root_b_ref, rule_W_ref, rule_b_ref,
               op_W_ref, op_b_ref, s_in_W_ref, s_in_b_ref,
               r1_W1_ref, r1_b1_ref, r1_W2_ref, r1_b2_ref,
               r2_W1_ref, r2_b1_ref, r2_W2_ref, r2_b2_ref,
               s_out_W_ref, s_out_b_ref,
               x_ref, root_ref, rule_ref, op_ref, split_ref):
    x = lax.dot_general(pred_ref[...], g_ref[...],
                        (((1,), (1,)), ((), ())),
                        preferred_element_type=jnp.float32)
    x = x + bv_ref[...].reshape(1, B) - lse_ref[...]
    x_ref[...] = x.reshape(1, QALL, B)

    @pl.when(pl.program_id(0) == 0)
    def _():
        root_ref[...] = _log_softmax_rows(root_W_ref[...] + root_b_ref[...])
        rule_ref[...] = _log_softmax_rows(rule_W_ref[...] + rule_b_ref[...])
        op_ref[...] = _log_softmax_rows(op_W_ref[...] + op_b_ref[...])
        pred = pred_ref[...]
        h = jnp.dot(pred, s_in_W_ref[...],
                    preferred_element_type=jnp.float32) + s_in_b_ref[...]
        t = jax.nn.relu(jnp.dot(h, r1_W1_ref[...],
                                preferred_element_type=jnp.float32)
                        + r1_b1_ref[...])
        h = h + jax.nn.relu(jnp.dot(t, r1_W2_ref[...],
                                    preferred_element_type=jnp.float32)
                            + r1_b2_ref[...])
        t = jax.nn.relu(jnp.dot(h, r2_W1_ref[...],
                                preferred_element_type=jnp.float32)
                        + r2_b1_ref[...])
        h = h + jax.nn.relu(jnp.dot(t, r2_W2_ref[...],
                                    preferred_element_type=jnp.float32)
                            + r2_b2_ref[...])
        sp = jnp.dot(h, s_out_W_ref[...],
                     preferred_element_type=jnp.float32) + s_out_b_ref[...]
        split_ref[...] = _log_softmax_rows(sp)


def _emit(g, bv2, predcat_emb, lse, root_W, root_b2, rule_W, rule_b2,
          op_W, op_b2, s_in_W, s_in_b2, r1_W1, r1_b12, r1_W2, r1_b22,
          r2_W1, r2_b12, r2_W2, r2_b22, s_out_W, s_out_b2):
    full = lambda shape: pl.BlockSpec(shape, lambda i: (0,) * len(shape))
    return pl.pallas_call(
        _emit_body,
        grid=(L,),
        in_specs=[
            pl.BlockSpec((B, STATE), lambda i: (i, 0)),
            pl.BlockSpec((1, 1, B), lambda i: (i, 0, 0)),
            full((QALL, STATE)),
            full((QALL, 1)),
            full((1, QALL)), full((1, QALL)),
            full(rule_W.shape), full((1, rule_W.shape[1])),
            full(op_W.shape), full((1, op_W.shape[1])),
            full(s_in_W.shape), full((1, STATE)),
            full(r1_W1.shape), full((1, STATE)),
            full(r1_W2.shape), full((1, STATE)),
            full(r2_W1.shape), full((1, STATE)),
            full(r2_W2.shape), full((1, STATE)),
            full(s_out_W.shape), full((1, s_out_W.shape[1])),
        ],
        out_specs=[
            pl.BlockSpec((1, QALL, B), lambda i: (i, 0, 0)),
            full((1, QALL)),
            full(rule_W.shape),
            full(op_W.shape),
            full((QALL, s_out_W.shape[1])),
        ],
        out_shape=[
            jax.ShapeDtypeStruct((L, QALL, B), jnp.float32),
            jax.ShapeDtypeStruct((1, QALL), jnp.float32),
            jax.ShapeDtypeStruct(rule_W.shape, jnp.float32),
            jax.ShapeDtypeStruct(op_W.shape, jnp.float32),
            jax.ShapeDtypeStruct((QALL, s_out_W.shape[1]), jnp.float32),
        ],
    )(g, bv2, predcat_emb, lse, root_W, root_b2, rule_W, rule_b2,
      op_W, op_b2, s_in_W, s_in_b2, r1_W1, r1_b12, r1_W2, r1_b22,
      r2_W1, r2_b12, r2_W2, r2_b22, s_out_W, s_out_b2)


def kernel(words, emit_W, emit_b, predcat_emb, root_W, root_b, rule_W, rule_b,
           op_W, op_b, s_in_W, s_in_b, r1_W1, r1_b1, r1_W2, r1_b2,
           r2_W1, r2_b1, r2_W2, r2_b2, s_out_W, s_out_b):
    emit_wt = emit_W.T
    lse = _lse(predcat_emb, emit_wt, emit_b.reshape(1, VOCAB))
    words_flat = words.T.reshape(NWORDS).astype(jnp.int32)
    g, bv = _sc_gather(emit_wt, emit_b, words_flat)
    x_p, root, rule, op, split = _emit(
        g, bv.reshape(L, 1, B), predcat_emb, lse,
        root_W, root_b.reshape(1, QALL), rule_W, rule_b.reshape(1, -1),
        op_W, op_b.reshape(1, -1), s_in_W, s_in_b.reshape(1, -1),
        r1_W1, r1_b1.reshape(1, -1), r1_W2, r1_b2.reshape(1, -1),
        r2_W1, r2_b1.reshape(1, -1), r2_W2, r2_b2.reshape(1, -1),
        s_out_W, s_out_b.reshape(1, -1))
    x = jnp.transpose(x_p, (2, 0, 1))
    return (x, root.reshape(QALL), rule, op, split)
